# Initial kernel scaffold; baseline (speedup 1.0000x reference)
import jax
import jax.numpy as jnp
from jax.experimental import pallas as pl


def kernel(x, edge_index, edge_attr, brdf_coeffs):
    raise NotImplementedError("write your pallas kernel here")


# speedup vs baseline: 3.7745x; 3.7745x over previous
import functools
import math

import jax
import jax.numpy as jnp
from jax import lax
from jax.experimental import pallas as pl
from jax.experimental.pallas import tpu as pltpu
from jax.experimental.pallas import tpu_sc as plsc

N = 10000
E = 320000
D = 128
NUM_BRDFS = 4
NUM_BOUNCES = 8
FSM_GAMMA = 1e-3
SPEED_OF_SOUND = 343.0
MEAN_FREE_PATH = 5.0
AIR_ABS = 1e-3

WSCALE = math.exp(-AIR_ABS * MEAN_FREE_PATH) / 64.0
DECAY = math.exp(math.log(FSM_GAMMA) * (MEAN_FREE_PATH / SPEED_OF_SOUND))

NC = 2
NS = 16
L = 16
DH = D // NC
B = 128
NPAD = 10240
RPT = NPAD // NS
RCH = RPT // B
EPAD = 321536
EPT = EPAD // NS
NB = EPT // B
NROWS2 = NC * NPAD


def _sc_body(x_hbm, row_hbm, col_hbm, attr_hbm, coef_hbm,
             out_hbm, r_hbm,
             racc, row_r, col_r, w_r, gbuf, obuf, zbuf, abuf, coef_v, sem):
    c = lax.axis_index("c")
    s = lax.axis_index("s")
    rbase = c * NPAD + s * RPT
    lbase = s * RPT

    pltpu.sync_copy(coef_hbm, coef_v)
    pltpu.sync_copy(row_hbm.at[s], row_r)
    pltpu.sync_copy(col_hbm.at[s], col_r)

    cvec = coef_v[pl.ds(0, L)]
    c0 = cvec[0] * WSCALE
    c1 = cvec[1] * WSCALE
    c2 = cvec[2] * WSCALE
    c3 = cvec[3] * WSCALE
    coff = (c * NPAD).astype(jnp.int32)

    def init_batch(b, _):
        for k in range(NUM_BRDFS):
            pltpu.sync_copy(attr_hbm.at[pl.ds(k * EPAD + s * EPT + b * B, B)],
                            abuf.at[k])
        for i in range(B // L):
            sl = pl.ds(i * L, L)
            w = (abuf[0, sl] * c0 + abuf[1, sl] * c1
                 + abuf[2, sl] * c2 + abuf[3, sl] * c3)
            w_r[b, sl] = w
            row_r[b, sl] = row_r[b, sl] + coff
        return 0

    lax.fori_loop(0, NB, init_batch, 0, unroll=False)

    def zero_body(e, _):
        for f in range(DH // L):
            zbuf[e, pl.ds(f * L, L)] = jnp.zeros((L,), jnp.float32)
        return 0

    lax.fori_loop(0, B, zero_body, 0, unroll=False)

    def init_rows(j, _):
        pltpu.sync_copy(x_hbm.at[pl.ds(rbase + j * B, B)], gbuf)
        pltpu.sync_copy(gbuf, r_hbm.at[pl.ds(rbase + j * B, B)])
        pltpu.sync_copy(gbuf, out_hbm.at[pl.ds(rbase + j * B, B)])
        pltpu.sync_copy(zbuf, racc.at[pl.ds(lbase + j * B, B)])
        return 0

    lax.fori_loop(0, RCH, init_rows, 0, unroll=False)
    plsc.subcore_barrier()

    def bounce(t, _):
        def batch_body(b, _):
            pltpu.async_copy(r_hbm.at[row_r.at[b]], gbuf, sem).wait()

            def group_body(g, _):
                wv = w_r[b, pl.ds(g * L, L)]
                for j in range(L):
                    wj = jnp.full((L,), wv[j], jnp.float32)
                    e = g * L + j
                    for f in range(DH // L):
                        sl = pl.ds(f * L, L)
                        gbuf[e, sl] = gbuf[e, sl] * wj
                return 0

            lax.fori_loop(0, B // L, group_body, 0, unroll=False)
            pltpu.sync_copy(gbuf, racc.at[col_r.at[b]], add=True)
            return 0

        lax.fori_loop(0, NB, batch_body, 0, unroll=False)
        plsc.subcore_barrier()

        def chunk_body(j, _):
            pltpu.sync_copy(racc.at[pl.ds(lbase + j * B, B)], gbuf)
            pltpu.sync_copy(out_hbm.at[pl.ds(rbase + j * B, B)], obuf)

            def row_body(e, _):
                for f in range(DH // L):
                    sl = pl.ds(f * L, L)
                    v = gbuf[e, sl] * DECAY
                    gbuf[e, sl] = v
                    obuf[e, sl] = obuf[e, sl] + v
                return 0

            lax.fori_loop(0, B, row_body, 0, unroll=False)
            pltpu.sync_copy(gbuf, r_hbm.at[pl.ds(rbase + j * B, B)])
            pltpu.sync_copy(obuf, out_hbm.at[pl.ds(rbase + j * B, B)])
            pltpu.sync_copy(zbuf, racc.at[pl.ds(lbase + j * B, B)])
            return 0

        lax.fori_loop(0, RCH, chunk_body, 0, unroll=False)
        plsc.subcore_barrier()
        return 0

    lax.fori_loop(0, NUM_BOUNCES, bounce, 0, unroll=False)


def kernel(x, edge_index, edge_attr, brdf_coeffs):
    x2 = x.reshape(N, NC, DH).transpose(1, 0, 2)
    x2 = jnp.pad(x2, ((0, 0), (0, NPAD - N), (0, 0))).reshape(NROWS2, DH)

    row = jnp.pad(edge_index[0].astype(jnp.int32), (0, EPAD - E))
    col = jnp.pad(edge_index[1].astype(jnp.int32), (0, EPAD - E))
    row2 = row.reshape(NS, NB, B)
    col2 = col.reshape(NS, NB, B)
    attr2 = jnp.pad(edge_attr.astype(jnp.float32).T,
                    ((0, 0), (0, EPAD - E))).reshape(NUM_BRDFS * EPAD)
    coef = jnp.pad(brdf_coeffs.astype(jnp.float32), (0, L - NUM_BRDFS))

    mesh = plsc.VectorSubcoreMesh(core_axis_name="c", subcore_axis_name="s",
                                  num_cores=NC, num_subcores=NS)
    f32 = jnp.float32
    run = pl.kernel(
        _sc_body,
        out_type=(jax.ShapeDtypeStruct((NROWS2, DH), f32),
                  jax.ShapeDtypeStruct((NROWS2, DH), f32)),
        mesh=mesh,
        scratch_types=[
            pltpu.VMEM_SHARED((NPAD, DH), f32),
            pltpu.VMEM((NB, B), jnp.int32),
            pltpu.VMEM((NB, B), jnp.int32),
            pltpu.VMEM((NB, B), f32),
            pltpu.VMEM((B, DH), f32),
            pltpu.VMEM((B, DH), f32),
            pltpu.VMEM((B, DH), f32),
            pltpu.VMEM((NUM_BRDFS, B), f32),
            pltpu.VMEM((L,), f32),
            pltpu.SemaphoreType.DMA,
        ],
        compiler_params=pltpu.CompilerParams(use_tc_tiling_on_sc=False),
    )
    out2, _ = run(x2, row2, col2, attr2, coef)
    out = out2.reshape(NC, NPAD, DH)[:, :N]
    return out.transpose(1, 0, 2).reshape(N, D)


# speedup vs baseline: 3.9046x; 1.0345x over previous
import math

import jax
import jax.numpy as jnp
from jax import lax
from jax.experimental import pallas as pl
from jax.experimental.pallas import tpu as pltpu
from jax.experimental.pallas import tpu_sc as plsc

N = 10000
E = 320000
D = 128
NUM_BRDFS = 4
NUM_BOUNCES = 8
FSM_GAMMA = 1e-3
SPEED_OF_SOUND = 343.0
MEAN_FREE_PATH = 5.0
AIR_ABS = 1e-3

WSCALE = math.exp(-AIR_ABS * MEAN_FREE_PATH) / 64.0
DECAY = math.exp(math.log(FSM_GAMMA) * (MEAN_FREE_PATH / SPEED_OF_SOUND))

NC = 2
NS = 16
L = 16
DH = D // NC
B = 128
NPAD = 10240
RPT = NPAD // NS
RCH = RPT // B
EPAD = 327680
EPT = EPAD // NS
NB = EPT // B
NROWS2 = NC * NPAD


def _sc_body(x_hbm, row_hbm, col_hbm, attr_hbm, coef_hbm,
             out_hbm, r_hbm,
             racc, row_r, col_r, w_r,
             g0, g1, g2, g3, abuf, coef_v,
             sg0, sg1, sg2, sg3, ss0, ss1, ss2, ss3):
    gb = (g0, g1, g2, g3)
    sg = (sg0, sg1, sg2, sg3)
    ss = (ss0, ss1, ss2, ss3)
    c = lax.axis_index("c")
    s = lax.axis_index("s")
    rbase = c * NPAD + s * RPT
    lbase = s * RPT

    pltpu.sync_copy(coef_hbm, coef_v)
    pltpu.sync_copy(row_hbm.at[s], row_r)
    pltpu.sync_copy(col_hbm.at[s], col_r)

    cvec = coef_v[pl.ds(0, L)]
    c0 = cvec[0] * WSCALE
    c1 = cvec[1] * WSCALE
    c2 = cvec[2] * WSCALE
    c3 = cvec[3] * WSCALE
    coff = (c * NPAD).astype(jnp.int32)

    def init_batch(b, _):
        for k in range(NUM_BRDFS):
            pltpu.sync_copy(attr_hbm.at[pl.ds(k * EPAD + s * EPT + b * B, B)],
                            abuf.at[k])
        for h in range(B // (2 * L)):
            lo = pl.ds(h * 2 * L, L)
            hi = pl.ds(h * 2 * L + L, L)
            wlo = (abuf[0, lo] * c0 + abuf[1, lo] * c1
                   + abuf[2, lo] * c2 + abuf[3, lo] * c3)
            whi = (abuf[0, hi] * c0 + abuf[1, hi] * c1
                   + abuf[2, hi] * c2 + abuf[3, hi] * c3)
            w_r[b, pl.ds(h * 2 * L, 2 * L)] = plsc.pack(
                wlo, whi, format=plsc.PackFormat.INTERLEAVED)
            row_r[b, lo] = row_r[b, lo] + coff
            row_r[b, hi] = row_r[b, hi] + coff
        return 0

    lax.fori_loop(0, NB, init_batch, 0, unroll=False)

    def init_rows(j, _):
        pltpu.sync_copy(x_hbm.at[pl.ds(rbase + j * B, B)], g0)
        pltpu.sync_copy(g0, r_hbm.at[pl.ds(rbase + j * B, B)])
        pltpu.sync_copy(g0, out_hbm.at[pl.ds(rbase + j * B, B)])

        def zz(e, _):
            for f in range(DH // L):
                g0[e, pl.ds(f * L, L)] = jnp.zeros((L,), jnp.float32)
            return 0

        lax.fori_loop(0, B, zz, 0, unroll=False)
        pltpu.sync_copy(g0, racc.at[pl.ds(lbase + j * B, B)])
        return 0

    lax.fori_loop(0, RCH, init_rows, 0, unroll=False)
    plsc.subcore_barrier()

    def bounce(t, _):
        pltpu.async_copy(racc.at[pl.ds(0, B)], gb[2], ss[2])
        pltpu.async_copy(racc.at[pl.ds(0, B)], gb[3], ss[3])
        pltpu.async_copy(r_hbm.at[row_r.at[0]], gb[0], sg[0])
        pltpu.async_copy(r_hbm.at[row_r.at[1]], gb[1], sg[1])

        def quad_body(hq, _):
            for p in range(4):
                j = hq * 4 + p
                q = (p + 2) % 4
                buf = gb[p]
                pltpu.make_async_copy(r_hbm.at[row_r.at[j]], buf,
                                      sg[p]).wait()

                def mult_h(h, _):
                    wlo, whi = plsc.unpack(
                        w_r[j, pl.ds(h * 2 * L, 2 * L)],
                        format=plsc.PackFormat.INTERLEAVED)
                    for k in range(2):
                        wv = (wlo, whi)[k]
                        for jj in range(L):
                            wj = jnp.full((L,), wv[jj], jnp.float32)
                            e = h * 2 * L + k * L + jj
                            for f in range(DH // L):
                                sl = pl.ds(f * L, L)
                                buf[e, sl] = buf[e, sl] * wj
                    return 0

                lax.fori_loop(0, B // (2 * L), mult_h, 0, unroll=False)
                pltpu.async_copy(buf, racc.at[col_r.at[j]], ss[p], add=True)
                pltpu.make_async_copy(gb[q], racc.at[col_r.at[j]],
                                      ss[q]).wait()
                jn = jnp.minimum(j + 2, NB - 1)
                pltpu.async_copy(r_hbm.at[row_r.at[jn]], gb[q], sg[q])
            return 0

        lax.fori_loop(0, NB // 4, quad_body, 0, unroll=False)
        pltpu.make_async_copy(r_hbm.at[row_r.at[0]], gb[0], sg[0]).wait()
        pltpu.make_async_copy(r_hbm.at[row_r.at[0]], gb[1], sg[1]).wait()
        pltpu.make_async_copy(gb[2], racc.at[col_r.at[0]], ss[2]).wait()
        pltpu.make_async_copy(gb[3], racc.at[col_r.at[0]], ss[3]).wait()
        plsc.subcore_barrier()

        def chunk_body(j, _):
            pltpu.sync_copy(racc.at[pl.ds(lbase + j * B, B)], g0)
            pltpu.sync_copy(out_hbm.at[pl.ds(rbase + j * B, B)], g1)

            def row_body(e, _):
                for f in range(DH // L):
                    sl = pl.ds(f * L, L)
                    v = g0[e, sl] * DECAY
                    g0[e, sl] = v
                    g1[e, sl] = g1[e, sl] + v
                return 0

            lax.fori_loop(0, B, row_body, 0, unroll=False)
            pltpu.sync_copy(g0, r_hbm.at[pl.ds(rbase + j * B, B)])
            pltpu.sync_copy(g1, out_hbm.at[pl.ds(rbase + j * B, B)])

            def zz(e, _):
                for f in range(DH // L):
                    g0[e, pl.ds(f * L, L)] = jnp.zeros((L,), jnp.float32)
                return 0

            lax.fori_loop(0, B, zz, 0, unroll=False)
            pltpu.sync_copy(g0, racc.at[pl.ds(lbase + j * B, B)])
            return 0

        lax.fori_loop(0, RCH, chunk_body, 0, unroll=False)
        plsc.subcore_barrier()
        return 0

    lax.fori_loop(0, NUM_BOUNCES, bounce, 0, unroll=False)


def kernel(x, edge_index, edge_attr, brdf_coeffs):
    x2 = x.reshape(N, NC, DH).transpose(1, 0, 2)
    x2 = jnp.pad(x2, ((0, 0), (0, NPAD - N), (0, 0))).reshape(NROWS2, DH)

    row = jnp.pad(edge_index[0].astype(jnp.int32), (0, EPAD - E))
    col = jnp.pad(edge_index[1].astype(jnp.int32), (0, EPAD - E))
    row2 = row.reshape(NS, NB, B)
    col2 = col.reshape(NS, NB, B)
    attr2 = jnp.pad(edge_attr.astype(jnp.float32).T,
                    ((0, 0), (0, EPAD - E))).reshape(NUM_BRDFS * EPAD)
    coef = jnp.pad(brdf_coeffs.astype(jnp.float32), (0, L - NUM_BRDFS))

    mesh = plsc.VectorSubcoreMesh(core_axis_name="c", subcore_axis_name="s",
                                  num_cores=NC, num_subcores=NS)
    f32 = jnp.float32
    run = pl.kernel(
        _sc_body,
        out_type=(jax.ShapeDtypeStruct((NROWS2, DH), f32),
                  jax.ShapeDtypeStruct((NROWS2, DH), f32)),
        mesh=mesh,
        scratch_types=[
            pltpu.VMEM_SHARED((NPAD, DH), f32),
            pltpu.VMEM((NB, B), jnp.int32),
            pltpu.VMEM((NB, B), jnp.int32),
            pltpu.VMEM((NB, B), jnp.bfloat16),
            pltpu.VMEM((B, DH), f32),
            pltpu.VMEM((B, DH), f32),
            pltpu.VMEM((B, DH), f32),
            pltpu.VMEM((B, DH), f32),
            pltpu.VMEM((NUM_BRDFS, B), f32),
            pltpu.VMEM((L,), f32),
            pltpu.SemaphoreType.DMA,
            pltpu.SemaphoreType.DMA,
            pltpu.SemaphoreType.DMA,
            pltpu.SemaphoreType.DMA,
            pltpu.SemaphoreType.DMA,
            pltpu.SemaphoreType.DMA,
            pltpu.SemaphoreType.DMA,
            pltpu.SemaphoreType.DMA,
        ],
        compiler_params=pltpu.CompilerParams(use_tc_tiling_on_sc=False,
                                             needs_layout_passes=False),
    )
    out2, _ = run(x2, row2, col2, attr2, coef)
    out = out2.reshape(NC, NPAD, DH)[:, :N]
    return out.transpose(1, 0, 2).reshape(N, D)
